# Initial kernel scaffold; baseline (speedup 1.0000x reference)
import jax
import jax.numpy as jnp
from jax.experimental import pallas as pl


def kernel(x, edge_idx, W, b):
    raise NotImplementedError("write your pallas kernel here")


# speedup vs baseline: 14.6535x; 14.6535x over previous
import functools

import jax
import jax.numpy as jnp
from jax import lax
from jax.experimental import pallas as pl
from jax.experimental.pallas import tpu as pltpu
from jax.experimental.pallas import tpu_sc as plsc

NC = 2
NS = 16
NW = NC * NS
LANES = 16
CHUNK = 128
DEG_W = 16


@functools.lru_cache(maxsize=None)
def _sc_degree(acc_rows, epw, n_chunks):
  mesh = plsc.VectorSubcoreMesh(core_axis_name="c", subcore_axis_name="s")
  stripe = acc_rows // NS

  def body(dst_hbm, out_hbm, deg_sh, ones_v, buf_v, dst_v):
    cid = lax.axis_index("c")
    sid = lax.axis_index("s")
    wid = sid * NC + cid
    ones16 = jnp.ones((LANES,), jnp.float32)
    zeros16 = jnp.zeros((LANES,), jnp.float32)
    for r in range(CHUNK):
      ones_v[r, :] = ones16
      buf_v[r, :] = zeros16

    def zero_blk(j, carry):
      pltpu.sync_copy(buf_v, deg_sh.at[pl.ds(sid * stripe + j * CHUNK, CHUNK)])
      return carry

    lax.fori_loop(0, stripe // CHUNK, zero_blk, 0)
    plsc.subcore_barrier()

    def edge_blk(i, carry):
      base = wid * epw + i * CHUNK
      pltpu.sync_copy(dst_hbm.at[pl.ds(base, CHUNK)], dst_v)
      pltpu.sync_copy(ones_v, deg_sh.at[dst_v], add=True)
      return carry

    lax.fori_loop(0, n_chunks, edge_blk, 0)
    plsc.subcore_barrier()

    def out_blk(j, carry):
      row = sid * stripe + j * CHUNK
      pltpu.sync_copy(deg_sh.at[pl.ds(row, CHUNK)], buf_v)
      pltpu.sync_copy(buf_v, out_hbm.at[cid, pl.ds(row, CHUNK)])
      return carry

    lax.fori_loop(0, stripe // CHUNK, out_blk, 0)

  return pl.kernel(
      body,
      out_type=jax.ShapeDtypeStruct((NC, acc_rows, DEG_W), jnp.float32),
      mesh=mesh,
      scratch_types=[
          pltpu.VMEM_SHARED((acc_rows, DEG_W), jnp.float32),
          pltpu.VMEM((CHUNK, DEG_W), jnp.float32),
          pltpu.VMEM((CHUNK, DEG_W), jnp.float32),
          pltpu.VMEM((CHUNK,), jnp.int32),
      ],
  )


@functools.lru_cache(maxsize=None)
def _sc_edge_agg(acc_rows, hdim, epw, n_chunks):
  mesh = plsc.VectorSubcoreMesh(core_axis_name="c", subcore_axis_name="s")
  stripe = acc_rows // NS

  def body(src_hbm, dst_hbm, hs_hbm, out_hbm,
           acc_sh, zbuf, src_v, dst_v, rows_v, sem):
    cid = lax.axis_index("c")
    sid = lax.axis_index("s")
    wid = sid * NC + cid
    zeros16 = jnp.zeros((LANES,), jnp.float32)
    for r in range(16):
      for k in range(hdim // LANES):
        zbuf[r, pl.ds(k * LANES, LANES)] = zeros16

    def zero_blk(j, carry):
      pltpu.sync_copy(zbuf, acc_sh.at[pl.ds(sid * stripe + j * 16, 16)])
      return carry

    lax.fori_loop(0, stripe // 16, zero_blk, 0)
    plsc.subcore_barrier()

    def edge_blk(i, carry):
      base = wid * epw + i * CHUNK
      pltpu.sync_copy(src_hbm.at[pl.ds(base, CHUNK)], src_v)
      pltpu.sync_copy(dst_hbm.at[pl.ds(base, CHUNK)], dst_v)
      pltpu.async_copy(hs_hbm.at[src_v], rows_v, sem).wait()
      pltpu.sync_copy(rows_v, acc_sh.at[dst_v], add=True)
      return carry

    lax.fori_loop(0, n_chunks, edge_blk, 0)
    plsc.subcore_barrier()

    def out_blk(j, carry):
      row = sid * stripe + j * CHUNK
      pltpu.sync_copy(acc_sh.at[pl.ds(row, CHUNK)], rows_v)
      pltpu.sync_copy(rows_v, out_hbm.at[cid, pl.ds(row, CHUNK)])
      return carry

    lax.fori_loop(0, stripe // CHUNK, out_blk, 0)

  return pl.kernel(
      body,
      out_type=jax.ShapeDtypeStruct((NC, acc_rows, hdim), jnp.float32),
      mesh=mesh,
      scratch_types=[
          pltpu.VMEM_SHARED((acc_rows, hdim), jnp.float32),
          pltpu.VMEM((16, hdim), jnp.float32),
          pltpu.VMEM((CHUNK,), jnp.int32),
          pltpu.VMEM((CHUNK,), jnp.int32),
          pltpu.VMEM((CHUNK, hdim), jnp.float32),
          pltpu.SemaphoreType.DMA,
      ],
  )


def _tc_linear(x, W, degp, blk):
  n, d = x.shape
  hdim = W.shape[1]

  def body(x_ref, w_ref, degp_ref, hs_ref):
    d2 = degp_ref[0] + degp_ref[1]
    dinv = lax.rsqrt(d2[:, :1] + 1.0)
    h = jnp.dot(x_ref[...], w_ref[...], preferred_element_type=jnp.float32)
    hs_ref[...] = h * dinv

  return pl.pallas_call(
      body,
      grid=(n // blk,),
      in_specs=[
          pl.BlockSpec((blk, d), lambda i: (i, 0)),
          pl.BlockSpec((d, hdim), lambda i: (0, 0)),
          pl.BlockSpec((NC, blk, DEG_W), lambda i: (0, i, 0)),
      ],
      out_specs=pl.BlockSpec((blk, hdim), lambda i: (i, 0)),
      out_shape=jax.ShapeDtypeStruct((n, hdim), jnp.float32),
  )(x, W, degp)


def _tc_finish(accp, hs, degp, b2, blk):
  n, hdim = hs.shape

  def body(accp_ref, hs_ref, degp_ref, b_ref, out_ref):
    acc = accp_ref[0] + accp_ref[1] + hs_ref[...]
    d2 = degp_ref[0] + degp_ref[1]
    dinv = lax.rsqrt(d2[:, :1] + 1.0)
    out_ref[...] = jnp.maximum(acc * dinv + b_ref[...], 0.0)

  return pl.pallas_call(
      body,
      grid=(n // blk,),
      in_specs=[
          pl.BlockSpec((NC, blk, hdim), lambda i: (0, i, 0)),
          pl.BlockSpec((blk, hdim), lambda i: (i, 0)),
          pl.BlockSpec((NC, blk, DEG_W), lambda i: (0, i, 0)),
          pl.BlockSpec((1, hdim), lambda i: (0, 0)),
      ],
      out_specs=pl.BlockSpec((blk, hdim), lambda i: (i, 0)),
      out_shape=jax.ShapeDtypeStruct((n, hdim), jnp.float32),
  )(accp, hs, degp, b2)


def kernel(x, edge_idx, W, b):
  n, _ = x.shape
  hdim = W.shape[1]
  e = edge_idx.shape[1]
  src = edge_idx[0].astype(jnp.int32)
  dst = edge_idx[1].astype(jnp.int32)

  n_chunks = -(-e // (NW * CHUNK))
  epw = n_chunks * CHUNK
  pad = epw * NW - e
  if pad:
    src = jnp.concatenate([src, jnp.zeros((pad,), jnp.int32)])
    dst = jnp.concatenate([dst, jnp.full((pad,), n, jnp.int32)])
  acc_rows = -(-(n + 1) // (NS * CHUNK)) * (NS * CHUNK)

  blk = 1000 if n % 1000 == 0 else n
  degp = _sc_degree(acc_rows, epw, n_chunks)(dst)
  hs = _tc_linear(x, W, degp, blk)
  accp = _sc_edge_agg(acc_rows, hdim, epw, n_chunks)(src, dst, hs)
  return _tc_finish(accp, hs, degp, b.reshape(1, hdim), blk)
